# Initial kernel scaffold; baseline (speedup 1.0000x reference)
import jax
import jax.numpy as jnp
from jax.experimental import pallas as pl


def kernel(token_ids, table):
    raise NotImplementedError("write your pallas kernel here")


# speedup vs baseline: 2.2784x; 2.2784x over previous
import functools

import jax
import jax.numpy as jnp
from jax import lax
from jax.experimental import pallas as pl
from jax.experimental.pallas import tpu as pltpu
from jax.experimental.pallas import tpu_sc as plsc

VOCAB = 50257
EMBED_DIM = 768
EPS = 1e-12

_NC = 2
_NS = 16
_NW = _NC * _NS

_CHUNK = 64


def _normalize_body(x_ref, o_ref):
    x = x_ref[...]
    ssq = jnp.sum(x * x, axis=1, keepdims=True)
    o_ref[...] = x / jnp.maximum(jnp.sqrt(ssq), EPS)


def _normalize_table(table):
    rows, d = table.shape
    br = 1024
    return pl.pallas_call(
        _normalize_body,
        grid=(pl.cdiv(rows, br),),
        in_specs=[pl.BlockSpec((br, d), lambda i: (i, 0))],
        out_specs=pl.BlockSpec((br, d), lambda i: (i, 0)),
        out_shape=jax.ShapeDtypeStruct((rows, d), table.dtype),
    )(table)


def _gather_body(n_chunks, ids_hbm, tab_hbm, out_hbm,
                 idx0, idx1, rows0, rows1, gsem0, gsem1, osem0, osem1):
    wid = lax.axis_index("s") * _NC + lax.axis_index("c")
    per_w = n_chunks * _CHUNK
    base = wid * per_w

    idx_l = (idx0, idx1)
    rows_l = (rows0, rows1)
    gsem_l = (gsem0, gsem1)
    osem_l = (osem0, osem1)

    def issue_gather(g, b):
        off = base + g * _CHUNK
        pltpu.sync_copy(ids_hbm.at[pl.ds(off, _CHUNK)], idx_l[b])
        pltpu.make_async_copy(tab_hbm.at[idx_l[b]], rows_l[b], gsem_l[b]).start()

    for b in range(2):
        issue_gather(b, b)

    def step(i, _):
        for b in range(2):
            g = i * 2 + b
            pltpu.make_async_copy(tab_hbm.at[idx_l[b]], rows_l[b], gsem_l[b]).wait()
            off = base + g * _CHUNK
            out_view = out_hbm.at[pl.ds(off, _CHUNK)]
            pltpu.make_async_copy(rows_l[b], out_view, osem_l[b]).start()
            pltpu.make_async_copy(rows_l[b], out_view, osem_l[b]).wait()

            nxt = g + 2

            @pl.when(nxt < n_chunks)
            def _():
                issue_gather(nxt, b)

        return _

    lax.fori_loop(0, n_chunks // 2, step, None)


def _gather_rows(table_n, flat_ids):
    b_tot = flat_ids.shape[0]
    d = table_n.shape[1]
    per_w = b_tot // _NW
    n_chunks = per_w // _CHUNK

    mesh = plsc.VectorSubcoreMesh(
        core_axis_name="c", subcore_axis_name="s",
        num_cores=_NC, num_subcores=_NS)

    grab = pl.kernel(
        functools.partial(_gather_body, n_chunks),
        out_type=jax.ShapeDtypeStruct((b_tot, d), jnp.float32),
        mesh=mesh,
        scratch_types=[
            pltpu.VMEM((_CHUNK,), jnp.int32),
            pltpu.VMEM((_CHUNK,), jnp.int32),
            pltpu.VMEM((_CHUNK, d), jnp.float32),
            pltpu.VMEM((_CHUNK, d), jnp.float32),
            pltpu.SemaphoreType.DMA,
            pltpu.SemaphoreType.DMA,
            pltpu.SemaphoreType.DMA,
            pltpu.SemaphoreType.DMA,
        ],
    )
    return grab(flat_ids, table_n)


def kernel(token_ids, table):
    bsz, seq = token_ids.shape
    table_n = _normalize_table(table)
    flat_ids = token_ids.reshape(-1).astype(jnp.int32)
    out = _gather_rows(table_n, flat_ids)
    return out.reshape(bsz, seq, EMBED_DIM)
